# baseline (speedup 1.0000x reference)
import functools

import jax
import jax.numpy as jnp
from jax import lax
from jax.experimental import pallas as pl
from jax.experimental.pallas import tpu as pltpu
from jax.experimental.pallas import tpu_sc as plsc

B = 16384
D = 64

_info = plsc.get_sparse_core_info()
NC, NS = _info.num_cores, _info.num_subcores
NW = NC * NS
BPW = B // NW
CHUNK = 128
NCHUNK = BPW // CHUNK


def _sc_gather(ridx2d, eidx2d, region_table, eid_table):
    mesh = plsc.VectorSubcoreMesh(core_axis_name="c", subcore_axis_name="s")

    @functools.partial(
        pl.kernel,
        out_type=(
            jax.ShapeDtypeStruct((B, D), jnp.float32),
            jax.ShapeDtypeStruct((B, D), jnp.float32),
        ),
        mesh=mesh,
        scratch_types=[
            pltpu.VMEM((NCHUNK, CHUNK), jnp.int32),
            pltpu.VMEM((NCHUNK, CHUNK), jnp.int32),
            pltpu.VMEM((BPW, D), jnp.float32),
            pltpu.VMEM((BPW, D), jnp.float32),
            pltpu.SemaphoreType.DMA,
        ],
        compiler_params=pltpu.CompilerParams(use_tc_tiling_on_sc=False),
    )
    def k(ridx_hbm, eidx_hbm, rtab_hbm, etab_hbm, r_out, e_out,
          ridx_v, eidx_v, rrows_v, erows_v, sem):
        wid = lax.axis_index("s") * NC + lax.axis_index("c")
        row0 = wid * NCHUNK
        pltpu.sync_copy(ridx_hbm.at[pl.ds(row0, NCHUNK)], ridx_v)
        pltpu.sync_copy(eidx_hbm.at[pl.ds(row0, NCHUNK)], eidx_v)
        copies = []
        for j in range(NCHUNK):
            copies.append(pltpu.async_copy(
                rtab_hbm.at[ridx_v.at[j]],
                rrows_v.at[pl.ds(j * CHUNK, CHUNK)], sem))
            copies.append(pltpu.async_copy(
                etab_hbm.at[eidx_v.at[j]],
                erows_v.at[pl.ds(j * CHUNK, CHUNK)], sem))
        for c in copies:
            c.wait()
        base = wid * BPW
        pltpu.sync_copy(rrows_v, r_out.at[pl.ds(base, BPW)])
        pltpu.sync_copy(erows_v, e_out.at[pl.ds(base, BPW)])

    return k(ridx2d, eidx2d, region_table, eid_table)


def _tc_body(r_ref, e_ref, g_ref, bt_ref, wt_ref, bias_ref, out_ref):
    h = jnp.concatenate([r_ref[...], e_ref[...]], axis=-1)
    mean = jnp.mean(h, axis=-1, keepdims=True)
    var = jnp.mean(jnp.square(h - mean), axis=-1, keepdims=True)
    hn = (h - mean) * lax.rsqrt(var + 1e-5)
    hn = hn * g_ref[...] + bt_ref[...]
    out_ref[...] = (
        jnp.dot(hn, wt_ref[...], preferred_element_type=jnp.float32)
        + bias_ref[...]
    )


def _tc_project(r, e, ln_gamma, ln_beta, W_T, b):
    BLK = 1024
    grid = (B // BLK,)
    return pl.pallas_call(
        _tc_body,
        grid=grid,
        in_specs=[
            pl.BlockSpec((BLK, D), lambda i: (i, 0)),
            pl.BlockSpec((BLK, D), lambda i: (i, 0)),
            pl.BlockSpec((1, 2 * D), lambda i: (0, 0)),
            pl.BlockSpec((1, 2 * D), lambda i: (0, 0)),
            pl.BlockSpec((2 * D, D), lambda i: (0, 0)),
            pl.BlockSpec((1, D), lambda i: (0, 0)),
        ],
        out_specs=pl.BlockSpec((BLK, D), lambda i: (i, 0)),
        out_shape=jax.ShapeDtypeStruct((B, D), jnp.float32),
    )(r, e, ln_gamma, ln_beta, W_T, b)


def kernel(region_ids, eids, region_table, eid_table, ln_gamma, ln_beta, W, b):
    ridx2d = region_ids.astype(jnp.int32).reshape(NW * NCHUNK, CHUNK)
    eidx2d = eids.astype(jnp.int32).reshape(NW * NCHUNK, CHUNK)
    r, e = _sc_gather(ridx2d, eidx2d, region_table, eid_table)
    return _tc_project(
        r, e,
        ln_gamma.reshape(1, 2 * D),
        ln_beta.reshape(1, 2 * D),
        W.T,
        b.reshape(1, D),
    )


# speedup vs baseline: 1.3920x; 1.3920x over previous
import functools

import jax
import jax.numpy as jnp
from jax import lax
from jax.experimental import pallas as pl
from jax.experimental.pallas import tpu as pltpu
from jax.experimental.pallas import tpu_sc as plsc

B = 16384
D = 64

_info = plsc.get_sparse_core_info()
NC, NS = _info.num_cores, _info.num_subcores
NW = NC * NS
BPW = B // NW
CHUNK = 128
NCHUNK = BPW // CHUNK


ROWS_PER_TILE = 8
C = 16
NCH = BPW // C


def _sc_gather(ridx, eidx, region_table, eid_table):
    mesh = plsc.VectorSubcoreMesh(core_axis_name="c", subcore_axis_name="s")

    @functools.partial(
        pl.kernel,
        out_type=(
            jax.ShapeDtypeStruct((B, D), jnp.float32),
            jax.ShapeDtypeStruct((B, D), jnp.float32),
        ),
        mesh=mesh,
        scratch_types=[
            pltpu.VMEM((BPW,), jnp.int32),
            pltpu.VMEM((BPW,), jnp.int32),
            pltpu.VMEM((C * ROWS_PER_TILE, D), jnp.float32),
            pltpu.VMEM((C * ROWS_PER_TILE, D), jnp.float32),
            pltpu.VMEM((C, D), jnp.float32),
            pltpu.VMEM((C, D), jnp.float32),
            pltpu.SemaphoreType.DMA,
            pltpu.SemaphoreType.DMA,
        ],
        compiler_params=pltpu.CompilerParams(needs_layout_passes=False),
    )
    def k(ridx_hbm, eidx_hbm, rtab_hbm, etab_hbm, r_out, e_out,
          ridx_v, eidx_v, rtile, etile, rext, eext,
          rsem, esem):
        wid = lax.axis_index("s") * NC + lax.axis_index("c")
        base = wid * BPW
        pltpu.sync_copy(ridx_hbm.at[pl.ds(base, BPW)], ridx_v)
        pltpu.sync_copy(eidx_hbm.at[pl.ds(base, BPW)], eidx_v)
        lane = lax.iota(jnp.int32, 16)

        @pl.loop(0, NCH)
        def _(ch):
            i0 = ch * C
            rvec = ridx_v[pl.ds(i0, C)]
            evec = eidx_v[pl.ds(i0, C)]
            rscal = [jnp.sum(jnp.where(lane == r, rvec, 0)) for r in range(C)]
            escal = [jnp.sum(jnp.where(lane == r, evec, 0)) for r in range(C)]
            for r in range(C):
                rt = pl.multiple_of(
                    (rscal[r] >> 3) * ROWS_PER_TILE, ROWS_PER_TILE)
                et = pl.multiple_of(
                    (escal[r] >> 3) * ROWS_PER_TILE, ROWS_PER_TILE)
                pltpu.async_copy(
                    rtab_hbm.at[pl.ds(rt, ROWS_PER_TILE)],
                    rtile.at[pl.ds(r * ROWS_PER_TILE, ROWS_PER_TILE)], rsem)
                pltpu.async_copy(
                    etab_hbm.at[pl.ds(et, ROWS_PER_TILE)],
                    etile.at[pl.ds(r * ROWS_PER_TILE, ROWS_PER_TILE)], esem)
            pltpu.make_async_copy(
                rtab_hbm.at[pl.ds(0, C * ROWS_PER_TILE)], rtile, rsem).wait()
            pltpu.make_async_copy(
                etab_hbm.at[pl.ds(0, C * ROWS_PER_TILE)], etile, esem).wait()
            for r in range(C):
                rrow = r * ROWS_PER_TILE + (rscal[r] & 7)
                erow = r * ROWS_PER_TILE + (escal[r] & 7)
                for j in range(D // 16):
                    sl = pl.ds(j * 16, 16)
                    rext[r, sl] = rtile[rrow, sl]
                    eext[r, sl] = etile[erow, sl]
            pltpu.sync_copy(rext, r_out.at[pl.ds(base + i0, C)])
            pltpu.sync_copy(eext, e_out.at[pl.ds(base + i0, C)])

    return k(ridx, eidx, region_table, eid_table)


def _tc_body(r_ref, e_ref, g_ref, bt_ref, wt_ref, bias_ref, out_ref):
    h = jnp.concatenate([r_ref[...], e_ref[...]], axis=-1)
    mean = jnp.mean(h, axis=-1, keepdims=True)
    var = jnp.mean(jnp.square(h - mean), axis=-1, keepdims=True)
    hn = (h - mean) * lax.rsqrt(var + 1e-5)
    hn = hn * g_ref[...] + bt_ref[...]
    out_ref[...] = (
        jnp.dot(hn, wt_ref[...], preferred_element_type=jnp.float32)
        + bias_ref[...]
    )


def _tc_project(r, e, ln_gamma, ln_beta, W_T, b):
    BLK = 1024
    grid = (B // BLK,)
    return pl.pallas_call(
        _tc_body,
        grid=grid,
        in_specs=[
            pl.BlockSpec((BLK, D), lambda i: (i, 0)),
            pl.BlockSpec((BLK, D), lambda i: (i, 0)),
            pl.BlockSpec((1, 2 * D), lambda i: (0, 0)),
            pl.BlockSpec((1, 2 * D), lambda i: (0, 0)),
            pl.BlockSpec((2 * D, D), lambda i: (0, 0)),
            pl.BlockSpec((1, D), lambda i: (0, 0)),
        ],
        out_specs=pl.BlockSpec((BLK, D), lambda i: (i, 0)),
        out_shape=jax.ShapeDtypeStruct((B, D), jnp.float32),
    )(r, e, ln_gamma, ln_beta, W_T, b)


def kernel(region_ids, eids, region_table, eid_table, ln_gamma, ln_beta, W, b):
    r, e = _sc_gather(region_ids.astype(jnp.int32), eids.astype(jnp.int32),
                      region_table, eid_table)
    return _tc_project(
        r, e,
        ln_gamma.reshape(1, 2 * D),
        ln_beta.reshape(1, 2 * D),
        W.T,
        b.reshape(1, D),
    )
